# baseline (speedup 1.0000x reference)
import functools

import jax
import jax.numpy as jnp
import numpy as np
from jax import lax
from jax.experimental import pallas as pl
from jax.experimental.pallas import tpu as pltpu
from jax.experimental.pallas import tpu_sc as plsc

_NC = 8
_NB = 16
_LAYERS = ((64, 64), (32, 32), (16, 16))
_ANC = np.array([[[10., 13.], [16., 30.], [33., 23.]],
                 [[30., 61.], [62., 45.], [59., 119.]],
                 [[116., 90.], [156., 198.], [373., 326.]]], dtype=np.float32)
_NT = 64
_EPS = 1e-7


def _sc_body(p0, p1, p2, tgt, ind, out, tgt_v, ind_v, best_v, idx0_v, idx1_v,
             idx2_v, b0_v, b1_v, b2_v, sem):
    cix = lax.axis_index("c")
    six = lax.axis_index("s")

    @pl.when(jnp.logical_and(cix == 0, six == 0))
    def _():
        pltpu.sync_copy(tgt, tgt_v)
        pltpu.sync_copy(ind, ind_v)
        lane = lax.iota(jnp.int32, 16)
        zeros = lane * 0
        d0 = plsc.load_gather(ind_v, [zeros])
        d1 = plsc.load_gather(ind_v, [zeros + 1])
        idxs = (idx0_v, idx1_v, idx2_v)
        for k in range(_NT // 16):
            gidx = lane + 16 * k
            b6 = gidx * 6
            tbf = plsc.load_gather(tgt_v, [b6])
            xr = plsc.load_gather(tgt_v, [b6 + 2])
            yr = plsc.load_gather(tgt_v, [b6 + 3])
            wr = plsc.load_gather(tgt_v, [b6 + 4])
            hr = plsc.load_gather(tgt_v, [b6 + 5])
            tb = tbf.astype(jnp.int32)
            gwd = wr * d0
            ght = hr * d1
            for l, (gh, gw) in enumerate(_LAYERS):
                gx = xr * float(gw)
                gy = yr * float(gh)
                gi = jnp.minimum(jnp.maximum(gx.astype(jnp.int32), 0), gw - 1)
                gj = jnp.minimum(jnp.maximum(gy.astype(jnp.int32), 0), gh - 1)
                us = []
                for a in range(3):
                    aw = float(_ANC[l, a, 0]) * gw
                    ah = float(_ANC[l, a, 1]) * gh
                    inter = jnp.minimum(gwd, aw) * jnp.minimum(ght, ah)
                    union = gwd * ght + (aw * ah) - inter + 1e-16
                    us.append(inter / union)
                b0 = jnp.logical_and(us[0] >= us[1], us[0] >= us[2])
                best = jnp.where(b0, 0, jnp.where(us[1] >= us[2], 1, 2))
                best = best.astype(jnp.int32)
                e0 = ((tb * 39 + best * 13) * gh + gj) * gw + gi
                for ch in range(13):
                    plsc.store_scatter(idxs[l], [gidx * 16 + ch],
                                       e0 + ch * (gh * gw))
                for ch in (13, 14, 15):
                    plsc.store_scatter(idxs[l], [gidx * 16 + ch], e0)
                plsc.store_scatter(best_v, [l * _NT + gidx],
                                   best.astype(jnp.float32))
        bufs = (b0_v, b1_v, b2_v)
        copies = []
        for pref, idxl, bufl in zip((p0, p1, p2), idxs, bufs):
            copies.append(pltpu.async_copy(pref.at[idxl], bufl, sem))
        for cp in copies:
            cp.wait()
        for l, bufl in enumerate(bufs):
            for k in range(_NT // 16):
                gidx = lane + 16 * k
                bv = plsc.load_gather(best_v, [l * _NT + gidx])
                plsc.store_scatter(bufl, [gidx * 16 + 13], bv)
            pltpu.sync_copy(bufl, out.at[pl.ds(l * _NT * 16, _NT * 16)])


@functools.cache
def _sc_gather():
    return pl.kernel(
        _sc_body,
        out_type=jax.ShapeDtypeStruct((3 * _NT * 16,), jnp.float32),
        mesh=plsc.VectorSubcoreMesh(core_axis_name="c", subcore_axis_name="s"),
        compiler_params=pltpu.CompilerParams(needs_layout_passes=False,
                                             use_tc_tiling_on_sc=True),
        scratch_types=[
            pltpu.VMEM((_NT * 6,), jnp.float32),
            pltpu.VMEM((16,), jnp.float32),
            pltpu.VMEM((3 * _NT,), jnp.float32),
            pltpu.VMEM((_NT * 16,), jnp.int32),
            pltpu.VMEM((_NT * 16,), jnp.int32),
            pltpu.VMEM((_NT * 16,), jnp.int32),
            pltpu.VMEM((_NT * 16,), jnp.float32),
            pltpu.VMEM((_NT * 16,), jnp.float32),
            pltpu.VMEM((_NT * 16,), jnp.float32),
            pltpu.SemaphoreType.DMA,
        ],
    )


def _tc_body(ind_s, p0_ref, p1_ref, p2_ref, tgt_ref, sc_ref, out_ref, acc):
    a = pl.program_id(0)

    @pl.when(a == 0)
    def _():
        for l in range(3):
            acc[l] = 0.0

    for l, pref in enumerate((p0_ref, p1_ref, p2_ref)):
        z = pref[:, 0]
        conf = jnp.clip(jax.nn.sigmoid(z), _EPS, 1.0 - _EPS)
        acc[l] = acc[l] + jnp.sum(-jnp.log(1.0 - conf))

    @pl.when(a == 2)
    def _():
        tgt = tgt_ref[...]
        tbf = tgt[:, 0:1]
        tclf = tgt[:, 1:2]
        xr = tgt[:, 2:3]
        yr = tgt[:, 3:4]
        wr = tgt[:, 4:5]
        hr = tgt[:, 5:6]
        d0 = ind_s[0]
        d1 = ind_s[1]
        tb = tbf
        gwd = wr * d0
        ght = hr * d1
        ii = lax.broadcasted_iota(jnp.int32, (_NT, _NT), 0)
        jj = lax.broadcasted_iota(jnp.int32, (_NT, _NT), 1)
        eye = (ii == jj).astype(jnp.float32)
        later = (jj > ii).astype(jnp.float32)
        total = 0.0
        for l, (gh, gw) in enumerate(_LAYERS):
            col = sc_ref[l]
            bestf = col[:, 13:14]
            gx = xr * float(gw)
            gy = yr * float(gh)
            gif = jnp.clip(jnp.floor(gx), 0.0, float(gw - 1))
            gjf = jnp.clip(jnp.floor(gy), 0.0, float(gh - 1))
            cid = ((tb * 3.0 + bestf) * gh + gjf) * gw + gif
            key2 = cid * float(_NC) + tclf
            live = None
            live2 = None
            masks = []
            for keyv in (cid, key2):
                kb = jnp.broadcast_to(keyv, (_NT, _NT))
                krow = jnp.sum(eye * kb, axis=0, keepdims=True)
                eq = (kb == jnp.broadcast_to(krow, (_NT, _NT))).astype(jnp.float32)
                dupcnt = jnp.sum(eq * later, axis=1, keepdims=True)
                masks.append((dupcnt == 0.0).astype(jnp.float32))
            live, live2 = masks
            nobj = jnp.sum(live)
            x = jax.nn.sigmoid(col[:, 0:1])
            y = jax.nn.sigmoid(col[:, 1:2])
            w = col[:, 2:3]
            h = col[:, 3:4]
            conf = jnp.clip(jax.nn.sigmoid(col[:, 4:5]), _EPS, 1.0 - _EPS)
            tx = gx - gif
            ty = gy - gjf
            aw0 = float(_ANC[l, 0, 0]); aw1 = float(_ANC[l, 1, 0]); aw2 = float(_ANC[l, 2, 0])
            ah0 = float(_ANC[l, 0, 1]); ah1 = float(_ANC[l, 1, 1]); ah2 = float(_ANC[l, 2, 1])
            ancw = jnp.where(bestf == 0.0, aw0, jnp.where(bestf == 1.0, aw1, aw2))
            anch = jnp.where(bestf == 0.0, ah0, jnp.where(bestf == 1.0, ah1, ah2))
            tw = jnp.log(gwd / ancw + 1e-16)
            th = jnp.log(ght / anch + 1e-16)
            sx = jnp.sum(live * (x - tx) ** 2)
            sy = jnp.sum(live * (y - ty) ** 2)
            sw = jnp.sum(live * (w - tw) ** 2)
            sh = jnp.sum(live * (h - th) ** 2)
            sobj = jnp.sum(live * -jnp.log(conf))
            scorr = jnp.sum(live * -jnp.log(1.0 - conf))
            s_allneg = 0.0
            ptc = 0.0
            for c in range(_NC):
                p = jnp.clip(jax.nn.sigmoid(col[:, 5 + c:6 + c]), _EPS, 1.0 - _EPS)
                s_allneg = s_allneg + jnp.sum(live * -jnp.log(1.0 - p))
                ptc = ptc + (tclf == float(c)).astype(jnp.float32) * p
            s_cls_corr = jnp.sum(live2 * (-jnp.log(ptc) + jnp.log(1.0 - ptc)))
            scls = s_allneg + s_cls_corr
            nd = jnp.maximum(nobj, 1.0)
            tot_l = float(_NB * 3 * gh * gw)
            total = total + (sx + sy + sw + sh + sobj) / nd \
                + 0.5 * (acc[l] - scorr) / jnp.maximum(tot_l - nobj, 1.0) \
                + scls / jnp.maximum(nobj * float(_NC), 1.0)
        out_ref[...] = jnp.broadcast_to(total, (1, 1))


def _tc_loss(ind, pred0, pred1, pred2, targets, scmat):
    specs = [pl.BlockSpec(memory_space=pltpu.SMEM)]
    for gh, gw in _LAYERS:
        specs.append(pl.BlockSpec((_NB, 1, gh, gw), lambda a: (0, 4 + 13 * a, 0, 0)))
    specs.append(pl.BlockSpec((_NT, 6), lambda a: (0, 0)))
    specs.append(pl.BlockSpec((3, _NT, 16), lambda a: (0, 0, 0)))
    return pl.pallas_call(
        _tc_body,
        grid=(3,),
        in_specs=specs,
        out_specs=pl.BlockSpec((1, 1), lambda a: (0, 0)),
        out_shape=jax.ShapeDtypeStruct((1, 1), jnp.float32),
        scratch_shapes=[pltpu.SMEM((3,), jnp.float32)],
    )(ind, pred0, pred1, pred2, targets, scmat)


def kernel(pred0, pred1, pred2, targets, input_dim):
    indf = jnp.asarray(input_dim, jnp.float32)
    ind16 = jnp.concatenate([indf, jnp.zeros((14,), jnp.float32)])
    scout = _sc_gather()(pred0.reshape(-1), pred1.reshape(-1),
                         pred2.reshape(-1), targets.reshape(-1), ind16)
    scmat = scout.reshape(3, _NT, 16)
    tot = _tc_loss(indf, pred0, pred1, pred2, targets, scmat)
    return tot[0, 0]


# speedup vs baseline: 1.5784x; 1.5784x over previous
import functools

import jax
import jax.numpy as jnp
import numpy as np
from jax import lax
from jax.experimental import pallas as pl
from jax.experimental.pallas import tpu as pltpu
from jax.experimental.pallas import tpu_sc as plsc

_NC = 8
_NB = 16
_LAYERS = ((64, 64), (32, 32), (16, 16))
_ANC = np.array([[[10., 13.], [16., 30.], [33., 23.]],
                 [[30., 61.], [62., 45.], [59., 119.]],
                 [[116., 90.], [156., 198.], [373., 326.]]], dtype=np.float32)
_NT = 64
_EPS = 1e-7
_NWORK = 32
_TPW = _NT // _NWORK


def _sc_body(p0, p1, p2, tgt, ind, out, tgt_v, ind_v, b00, b01, b02, b10,
             b11, b12, out0, out1, out2, sem):
    cix = lax.axis_index("c")
    six = lax.axis_index("s")
    wid = six * 2 + cix
    pltpu.sync_copy(tgt, tgt_v)
    pltpu.sync_copy(ind, ind_v)
    lane = lax.iota(jnp.int32, 16)
    zeros = lane * 0
    lane_c = jnp.minimum(lane, 12)
    d0 = plsc.load_gather(ind_v, [zeros])
    d1 = plsc.load_gather(ind_v, [zeros + 1])
    preds = (p0, p1, p2)
    bufs = ((b00, b01, b02), (b10, b11, b12))
    outs = (out0, out1, out2)
    copies = []
    gis = {}
    bests = {}
    for t_local in range(_TPW):
        t = wid * _TPW + t_local
        b6 = zeros + t * 6
        tbf = plsc.load_gather(tgt_v, [b6])
        xr = plsc.load_gather(tgt_v, [b6 + 2])
        yr = plsc.load_gather(tgt_v, [b6 + 3])
        wr = plsc.load_gather(tgt_v, [b6 + 4])
        hr = plsc.load_gather(tgt_v, [b6 + 5])
        tb = tbf.astype(jnp.int32)
        gwd = wr * d0
        ght = hr * d1
        for l, (gh, gw) in enumerate(_LAYERS):
            gx = xr * float(gw)
            gy = yr * float(gh)
            gi = jnp.minimum(jnp.maximum(gx.astype(jnp.int32), 0), gw - 1)
            gj = jnp.minimum(jnp.maximum(gy.astype(jnp.int32), 0), gh - 1)
            us = []
            for a in range(3):
                aw = float(_ANC[l, a, 0]) * gw
                ah = float(_ANC[l, a, 1]) * gh
                inter = jnp.minimum(gwd, aw) * jnp.minimum(ght, ah)
                union = gwd * ght + (aw * ah) - inter + 1e-16
                us.append(inter / union)
            b0 = jnp.logical_and(us[0] >= us[1], us[0] >= us[2])
            best = jnp.where(b0, 0, jnp.where(us[1] >= us[2], 1, 2))
            best = best.astype(jnp.int32)
            c0s = jnp.max(tb * 39 + best * 13)
            gjs = jnp.max(gj)
            copies.append(
                pltpu.async_copy(preds[l].at[pl.ds(c0s, 13), gjs],
                                 bufs[t_local][l].at[pl.ds(0, 13)], sem))
            gis[(t_local, l)] = gi
            bests[(t_local, l)] = best.astype(jnp.float32)
    for cp in copies:
        cp.wait()
    for t_local in range(_TPW):
        for l in range(3):
            v = plsc.load_gather(bufs[t_local][l],
                                 [lane_c, gis[(t_local, l)]])
            v = jnp.where(lane == 13, bests[(t_local, l)], v)
            plsc.store_scatter(outs[l], [lane + t_local * 16], v)
    for l in range(3):
        pltpu.sync_copy(outs[l],
                        out.at[pl.ds(l * _NT * 16 + wid * _TPW * 16,
                                     _TPW * 16)])


@functools.cache
def _sc_gather():
    return pl.kernel(
        _sc_body,
        out_type=jax.ShapeDtypeStruct((3 * _NT * 16,), jnp.float32),
        mesh=plsc.VectorSubcoreMesh(core_axis_name="c", subcore_axis_name="s"),
        compiler_params=pltpu.CompilerParams(needs_layout_passes=False,
                                             use_tc_tiling_on_sc=True),
        scratch_types=[
            pltpu.VMEM((_NT * 6,), jnp.float32),
            pltpu.VMEM((16,), jnp.float32),
            pltpu.VMEM((16, 64), jnp.float32),
            pltpu.VMEM((16, 32), jnp.float32),
            pltpu.VMEM((16, 16), jnp.float32),
            pltpu.VMEM((16, 64), jnp.float32),
            pltpu.VMEM((16, 32), jnp.float32),
            pltpu.VMEM((16, 16), jnp.float32),
            pltpu.VMEM((_TPW * 16,), jnp.float32),
            pltpu.VMEM((_TPW * 16,), jnp.float32),
            pltpu.VMEM((_TPW * 16,), jnp.float32),
            pltpu.SemaphoreType.DMA,
        ],
    )


def _tc_body(ind_s, p0_ref, p1_ref, p2_ref, tgt_ref, sc_ref, out_ref, acc):
    a = pl.program_id(0)

    @pl.when(a == 0)
    def _():
        for l in range(3):
            acc[l] = 0.0

    for l, pref in enumerate((p0_ref, p1_ref, p2_ref)):
        z = pref[:, 0]
        conf = jnp.clip(jax.nn.sigmoid(z), _EPS, 1.0 - _EPS)
        acc[l] = acc[l] + jnp.sum(-jnp.log(1.0 - conf))

    @pl.when(a == 2)
    def _():
        tgt = tgt_ref[...]
        tbf = tgt[:, 0:1]
        tclf = tgt[:, 1:2]
        xr = tgt[:, 2:3]
        yr = tgt[:, 3:4]
        wr = tgt[:, 4:5]
        hr = tgt[:, 5:6]
        d0 = ind_s[0]
        d1 = ind_s[1]
        tb = tbf
        gwd = wr * d0
        ght = hr * d1
        ii = lax.broadcasted_iota(jnp.int32, (_NT, _NT), 0)
        jj = lax.broadcasted_iota(jnp.int32, (_NT, _NT), 1)
        eye = (ii == jj).astype(jnp.float32)
        later = (jj > ii).astype(jnp.float32)
        total = 0.0
        for l, (gh, gw) in enumerate(_LAYERS):
            col = sc_ref[l]
            bestf = col[:, 13:14]
            gx = xr * float(gw)
            gy = yr * float(gh)
            gif = jnp.clip(jnp.floor(gx), 0.0, float(gw - 1))
            gjf = jnp.clip(jnp.floor(gy), 0.0, float(gh - 1))
            cid = ((tb * 3.0 + bestf) * gh + gjf) * gw + gif
            key2 = cid * float(_NC) + tclf
            live = None
            live2 = None
            masks = []
            for keyv in (cid, key2):
                kb = jnp.broadcast_to(keyv, (_NT, _NT))
                krow = jnp.sum(eye * kb, axis=0, keepdims=True)
                eq = (kb == jnp.broadcast_to(krow, (_NT, _NT))).astype(jnp.float32)
                dupcnt = jnp.sum(eq * later, axis=1, keepdims=True)
                masks.append((dupcnt == 0.0).astype(jnp.float32))
            live, live2 = masks
            nobj = jnp.sum(live)
            x = jax.nn.sigmoid(col[:, 0:1])
            y = jax.nn.sigmoid(col[:, 1:2])
            w = col[:, 2:3]
            h = col[:, 3:4]
            conf = jnp.clip(jax.nn.sigmoid(col[:, 4:5]), _EPS, 1.0 - _EPS)
            tx = gx - gif
            ty = gy - gjf
            aw0 = float(_ANC[l, 0, 0]); aw1 = float(_ANC[l, 1, 0]); aw2 = float(_ANC[l, 2, 0])
            ah0 = float(_ANC[l, 0, 1]); ah1 = float(_ANC[l, 1, 1]); ah2 = float(_ANC[l, 2, 1])
            ancw = jnp.where(bestf == 0.0, aw0, jnp.where(bestf == 1.0, aw1, aw2))
            anch = jnp.where(bestf == 0.0, ah0, jnp.where(bestf == 1.0, ah1, ah2))
            tw = jnp.log(gwd / ancw + 1e-16)
            th = jnp.log(ght / anch + 1e-16)
            sx = jnp.sum(live * (x - tx) ** 2)
            sy = jnp.sum(live * (y - ty) ** 2)
            sw = jnp.sum(live * (w - tw) ** 2)
            sh = jnp.sum(live * (h - th) ** 2)
            sobj = jnp.sum(live * -jnp.log(conf))
            scorr = jnp.sum(live * -jnp.log(1.0 - conf))
            s_allneg = 0.0
            ptc = 0.0
            for c in range(_NC):
                p = jnp.clip(jax.nn.sigmoid(col[:, 5 + c:6 + c]), _EPS, 1.0 - _EPS)
                s_allneg = s_allneg + jnp.sum(live * -jnp.log(1.0 - p))
                ptc = ptc + (tclf == float(c)).astype(jnp.float32) * p
            s_cls_corr = jnp.sum(live2 * (-jnp.log(ptc) + jnp.log(1.0 - ptc)))
            scls = s_allneg + s_cls_corr
            nd = jnp.maximum(nobj, 1.0)
            tot_l = float(_NB * 3 * gh * gw)
            total = total + (sx + sy + sw + sh + sobj) / nd \
                + 0.5 * (acc[l] - scorr) / jnp.maximum(tot_l - nobj, 1.0) \
                + scls / jnp.maximum(nobj * float(_NC), 1.0)
        out_ref[...] = jnp.broadcast_to(total, (1, 1))


def _tc_loss(ind, pred0, pred1, pred2, targets, scmat):
    specs = [pl.BlockSpec(memory_space=pltpu.SMEM)]
    for gh, gw in _LAYERS:
        specs.append(pl.BlockSpec((_NB, 1, gh, gw), lambda a: (0, 4 + 13 * a, 0, 0)))
    specs.append(pl.BlockSpec((_NT, 6), lambda a: (0, 0)))
    specs.append(pl.BlockSpec((3, _NT, 16), lambda a: (0, 0, 0)))
    return pl.pallas_call(
        _tc_body,
        grid=(3,),
        in_specs=specs,
        out_specs=pl.BlockSpec((1, 1), lambda a: (0, 0)),
        out_shape=jax.ShapeDtypeStruct((1, 1), jnp.float32),
        scratch_shapes=[pltpu.SMEM((3,), jnp.float32)],
    )(ind, pred0, pred1, pred2, targets, scmat)


def kernel(pred0, pred1, pred2, targets, input_dim):
    indf = jnp.asarray(input_dim, jnp.float32)
    ind16 = jnp.concatenate([indf, jnp.zeros((14,), jnp.float32)])
    p3d = [p.reshape(_NB * 39, gh, gw)
           for p, (gh, gw) in zip((pred0, pred1, pred2), _LAYERS)]
    scout = _sc_gather()(p3d[0], p3d[1], p3d[2], targets.reshape(-1), ind16)
    scmat = scout.reshape(3, _NT, 16)
    tot = _tc_loss(indf, pred0, pred1, pred2, targets, scmat)
    return tot[0, 0]
